# Initial kernel scaffold; baseline (speedup 1.0000x reference)
import jax
import jax.numpy as jnp
from jax.experimental import pallas as pl


def kernel(input_ids, attention_mask, x, edge_index, batch, node_emb, conv1_W, conv1_b, conv2_W, conv2_b, tok_emb, pos_emb, ln_e_g, ln_e_b, Wq, bq, Wk, bk, Wv, bv, Wo, bo, ln1_g, ln1_b, W1, b1, W2, b2, ln2_g, ln2_b, pool_W, pool_b, cls_W1, cls_b1, cls_W2, cls_b2):
    raise NotImplementedError("write your pallas kernel here")


# speedup vs baseline: 4.5783x; 4.5783x over previous
import functools

import jax
import jax.numpy as jnp
from jax import lax
from jax.experimental import pallas as pl
from jax.experimental.pallas import tpu as pltpu
from jax.experimental.pallas import tpu_sc as plsc

N_NODES = 10000
HID = 256
B = 32
S = 128
D = 768
H = 12
DH = 64
FF = 3072
NG = 32
N_EDGES = 160000

NC = 2
NS = 16
NW = NC * NS

RT = 640
R_ACC = RT * NS
E_TOT = N_EDGES + N_NODES
KE = 128
EB_TILE = 84
E_PAD = NS * KE * EB_TILE
EB_DEG = E_PAD // (NW * KE)
KN = 64
NODE_PB = 5
N_PAD = NW * KN * NODE_PB
KT = 64
TOK_PB = 2

_SC_MESH = plsc.VectorSubcoreMesh(core_axis_name="c", subcore_axis_name="s")


def _ln(x, g, b, eps=1e-12):
    m = x.mean(-1, keepdims=True)
    v = jnp.mean((x - m) ** 2, -1, keepdims=True)
    return (x - m) / jnp.sqrt(v + eps) * g + b


@functools.partial(
    pl.kernel,
    out_type=(
        jax.ShapeDtypeStruct((N_PAD, HID), jnp.float32),
        jax.ShapeDtypeStruct((B * S, D), jnp.float32),
    ),
    mesh=_SC_MESH,
    scratch_types=[
        pltpu.VMEM((KN,), jnp.int32),
        pltpu.VMEM((KN, HID), jnp.float32),
        pltpu.VMEM((KT,), jnp.int32),
        pltpu.VMEM((KT, D), jnp.float32),
        pltpu.SemaphoreType.DMA,
    ],
)
def _sc_gather(nidx_hbm, node_emb, tids_hbm, tok_emb, xe_out, tok_out,
               nidx_v, nrows_v, tidx_v, trows_v, sem):
    c = lax.axis_index("c")
    t = lax.axis_index("s")
    w = t * NC + c

    def node_blk(i, _):
        base = pl.multiple_of((w * NODE_PB + i) * KN, 8)
        pltpu.sync_copy(nidx_hbm.at[pl.ds(base, KN)], nidx_v)
        pltpu.async_copy(node_emb.at[nidx_v], nrows_v, sem).wait()
        pltpu.sync_copy(nrows_v, xe_out.at[pl.ds(base, KN)])
        return _
    lax.fori_loop(0, NODE_PB, node_blk, None)

    def tok_blk(i, _):
        base = pl.multiple_of((w * TOK_PB + i) * KT, 8)
        pltpu.sync_copy(tids_hbm.at[pl.ds(base, KT)], tidx_v)
        pltpu.async_copy(tok_emb.at[tidx_v], trows_v, sem).wait()
        pltpu.sync_copy(trows_v, tok_out.at[pl.ds(base, KT)])
        return _
    lax.fori_loop(0, TOK_PB, tok_blk, None)


@functools.partial(
    pl.kernel,
    out_type=jax.ShapeDtypeStruct((NC, R_ACC, 128), jnp.float32),
    mesh=_SC_MESH,
    scratch_types=[
        pltpu.VMEM((KE,), jnp.int32),
        pltpu.VMEM((KE,), jnp.int32),
        pltpu.VMEM((KE, 128), jnp.float32),
        pltpu.VMEM_SHARED((R_ACC, 128), jnp.float32),
        pltpu.SemaphoreType.DMA,
    ],
)
def _sc_msg_pass(hs_hbm, src2_hbm, dst_hbm, z128_hbm, scat_out,
                 sidx_v, didx_v, rows_v, acc, sem):
    c = lax.axis_index("c")
    t = lax.axis_index("s")
    pltpu.sync_copy(z128_hbm, rows_v)
    for j in range(RT // KE):
        base = pl.multiple_of(t * RT + j * KE, 8)
        pltpu.sync_copy(rows_v, acc.at[pl.ds(base, KE)])
    plsc.subcore_barrier()

    def edge_blk(i, _):
        base = pl.multiple_of(t * (KE * EB_TILE) + i * KE, 8)
        pltpu.sync_copy(src2_hbm.at[c, pl.ds(base, KE)], sidx_v)
        pltpu.sync_copy(dst_hbm.at[pl.ds(base, KE)], didx_v)
        pltpu.async_copy(hs_hbm.at[sidx_v], rows_v, sem).wait()
        pltpu.sync_copy(rows_v, acc.at[didx_v], add=True)
        return _
    lax.fori_loop(0, EB_TILE, edge_blk, None)

    plsc.subcore_barrier()
    for j in range(RT // KE):
        base = pl.multiple_of(t * RT + j * KE, 8)
        pltpu.sync_copy(acc.at[pl.ds(base, KE)], rows_v)
        pltpu.sync_copy(rows_v, scat_out.at[c, pl.ds(base, KE)])


BM = 1000


def _tc_hs1_body(xe_ref, w_ref, deg_ref, out_ref):
    dinv = lax.rsqrt(deg_ref[0][:, :1])
    h = jnp.dot(xe_ref[...], w_ref[...], preferred_element_type=jnp.float32)
    out_ref[...] = (h * dinv)[None]


def _tc_hs2_body(scat_ref, deg_ref, b1_ref, w_ref, out_ref):
    dinv = lax.rsqrt(deg_ref[0][:, :1])
    a = jnp.concatenate([scat_ref[0], scat_ref[1]], axis=1)
    h1 = jax.nn.relu(a * dinv + b1_ref[...])
    h = jnp.dot(h1, w_ref[...], preferred_element_type=jnp.float32)
    out_ref[...] = (h * dinv)[None]


def _tc_pool_body(scat_ref, deg_ref, b2_ref, batch_ref, seg_ref, cnt_ref):
    m = pl.program_id(0)
    dinv = lax.rsqrt(deg_ref[0][:, :1])
    a = jnp.concatenate([scat_ref[0], scat_ref[1]], axis=1)
    h2 = jax.nn.relu(a * dinv + b2_ref[...])
    bvec = batch_ref[0, 0, :]
    onehot = (bvec[:, None] == lax.broadcasted_iota(jnp.int32, (BM, NG), 1))
    onehot = onehot.astype(jnp.float32)
    contrib = lax.dot_general(onehot, h2, (((0,), (0,)), ((), ())),
                              preferred_element_type=jnp.float32)
    cntc = lax.dot_general(onehot, jnp.ones((BM, 128), jnp.float32),
                           (((0,), (0,)), ((), ())),
                           preferred_element_type=jnp.float32)

    @pl.when(m == 0)
    def _():
        seg_ref[...] = contrib
        cnt_ref[...] = cntc

    @pl.when(m > 0)
    def _():
        seg_ref[...] += contrib
        cnt_ref[...] += cntc


def _tc_attn_body(e_ref, pos_ref, mask_ref, lng_ref, lnb_ref,
                  wq_ref, bq_ref, wk_ref, bk_ref, wv_ref, bv_ref,
                  wo_ref, bo_ref, l1g_ref, l1b_ref, out_ref):
    x = e_ref[0] + pos_ref[...]
    hb0 = _ln(x, lng_ref[...], lnb_ref[...])
    q = jnp.dot(hb0, wq_ref[...], preferred_element_type=jnp.float32) + bq_ref[...]
    k = jnp.dot(hb0, wk_ref[...], preferred_element_type=jnp.float32) + bk_ref[...]
    v = jnp.dot(hb0, wv_ref[...], preferred_element_type=jnp.float32) + bv_ref[...]
    bias = jnp.where(mask_ref[0] > 0, 0.0, -1e9)
    scale = 1.0 / (DH ** 0.5)
    parts = []
    for h in range(H):
        qh = q[:, h * DH:(h + 1) * DH]
        kh = k[:, h * DH:(h + 1) * DH]
        vh = v[:, h * DH:(h + 1) * DH]
        s = lax.dot_general(qh, kh, (((1,), (1,)), ((), ())),
                            preferred_element_type=jnp.float32) * scale + bias
        att = jax.nn.softmax(s, axis=-1)
        parts.append(jnp.dot(att, vh, preferred_element_type=jnp.float32))
    ao = jnp.concatenate(parts, axis=1)
    proj = jnp.dot(ao, wo_ref[...], preferred_element_type=jnp.float32) + bo_ref[...]
    out_ref[...] = _ln(hb0 + proj, l1g_ref[...], l1b_ref[...])[None]


def _tc_ffn_body(hb_ref, w1_ref, b1_ref, w2_ref, b2_ref, l2g_ref, l2b_ref,
                 pw_ref, pb_ref, out_ref, acc_ref):
    kk = pl.program_id(1)
    x = hb_ref[0]
    tmid = jax.nn.gelu(jnp.dot(x, w1_ref[...],
                               preferred_element_type=jnp.float32) + b1_ref[...])
    c = jnp.dot(tmid, w2_ref[...], preferred_element_type=jnp.float32)

    @pl.when(kk == 0)
    def _():
        acc_ref[...] = c

    @pl.when(kk > 0)
    def _():
        acc_ref[...] += c

    @pl.when(kk == FF // D - 1)
    def _():
        hb2 = _ln(x + acc_ref[...] + b2_ref[...], l2g_ref[...], l2b_ref[...])
        cls_row = hb2[0:1, :]
        pooled = jnp.tanh(jnp.dot(cls_row, pw_ref[...],
                                  preferred_element_type=jnp.float32) + pb_ref[...])
        out_ref[...] = pooled[None]


def _tc_cls_body(seg_ref, cnt_ref, cls_ref, w1_ref, b1_ref, w2_ref, b2_ref,
                 out_ref):
    gf = seg_ref[...] / jnp.maximum(cnt_ref[:, :1], 1.0)
    fused = jnp.concatenate([gf, cls_ref[...]], axis=1)
    hid = jax.nn.relu(jnp.dot(fused, w1_ref[...],
                              preferred_element_type=jnp.float32) + b1_ref[...])
    out_ref[...] = jnp.dot(hid, w2_ref[...],
                           preferred_element_type=jnp.float32) + b2_ref[...]


def kernel(input_ids, attention_mask, x, edge_index, batch, node_emb, conv1_W,
           conv1_b, conv2_W, conv2_b, tok_emb, pos_emb, ln_e_g, ln_e_b, Wq, bq,
           Wk, bk, Wv, bv, Wo, bo, ln1_g, ln1_b, W1, b1, W2, b2, ln2_g, ln2_b,
           pool_W, pool_b, cls_W1, cls_b1, cls_W2, cls_b2):
    f32 = jnp.float32
    i32 = jnp.int32

    loop = jnp.arange(N_NODES, dtype=i32)
    src_f = jnp.concatenate([edge_index[0].astype(i32), loop])
    dst_f = jnp.concatenate([edge_index[1].astype(i32), loop])
    pad = E_PAD - E_TOT
    src_p = jnp.concatenate([src_f, jnp.zeros((pad,), i32)])
    dst_p = jnp.concatenate([dst_f, jnp.full((pad,), N_NODES, i32)])
    src2 = jnp.stack([src_p, src_p + N_NODES])
    nidx = jnp.concatenate([x[:, 0].astype(i32),
                            jnp.zeros((N_PAD - N_NODES,), i32)])
    tids = input_ids.reshape(B * S).astype(i32)
    z128 = jnp.zeros((KE, 128), f32)
    ones_flat = jnp.ones((NC * N_NODES, 128), f32)

    xe, etok = _sc_gather(nidx, node_emb, tids, tok_emb)
    deg2 = _sc_msg_pass(ones_flat, src2, dst_p, z128)

    hs1 = pl.pallas_call(
        _tc_hs1_body,
        grid=(N_NODES // BM, 2),
        in_specs=[
            pl.BlockSpec((BM, HID), lambda m, n: (m, 0)),
            pl.BlockSpec((HID, 128), lambda m, n: (0, n)),
            pl.BlockSpec((1, BM, 128), lambda m, n: (0, m, 0)),
        ],
        out_specs=pl.BlockSpec((1, BM, 128), lambda m, n: (n, m, 0)),
        out_shape=jax.ShapeDtypeStruct((NC, N_NODES, 128), f32),
    )(xe, conv1_W, deg2)

    scat1 = _sc_msg_pass(hs1.reshape(NC * N_NODES, 128), src2, dst_p, z128)

    hs2 = pl.pallas_call(
        _tc_hs2_body,
        grid=(N_NODES // BM, 2),
        in_specs=[
            pl.BlockSpec((NC, BM, 128), lambda m, n: (0, m, 0)),
            pl.BlockSpec((1, BM, 128), lambda m, n: (0, m, 0)),
            pl.BlockSpec((1, HID), lambda m, n: (0, 0)),
            pl.BlockSpec((HID, 128), lambda m, n: (0, n)),
        ],
        out_specs=pl.BlockSpec((1, BM, 128), lambda m, n: (n, m, 0)),
        out_shape=jax.ShapeDtypeStruct((NC, N_NODES, 128), f32),
    )(scat1, deg2, conv1_b.reshape(1, HID), conv2_W)

    scat2 = _sc_msg_pass(hs2.reshape(NC * N_NODES, 128), src2, dst_p, z128)

    batch3 = batch.astype(i32).reshape(N_NODES // BM, 1, BM)
    seg, cnt = pl.pallas_call(
        _tc_pool_body,
        grid=(N_NODES // BM,),
        in_specs=[
            pl.BlockSpec((NC, BM, 128), lambda m: (0, m, 0)),
            pl.BlockSpec((1, BM, 128), lambda m: (0, m, 0)),
            pl.BlockSpec((1, HID), lambda m: (0, 0)),
            pl.BlockSpec((1, 1, BM), lambda m: (m, 0, 0)),
        ],
        out_specs=(pl.BlockSpec((NG, HID), lambda m: (0, 0)),
                   pl.BlockSpec((NG, 128), lambda m: (0, 0))),
        out_shape=(jax.ShapeDtypeStruct((NG, HID), f32),
                   jax.ShapeDtypeStruct((NG, 128), f32)),
    )(scat2, deg2, conv2_b.reshape(1, HID), batch3)

    hb1 = pl.pallas_call(
        _tc_attn_body,
        grid=(B,),
        in_specs=[
            pl.BlockSpec((1, S, D), lambda b: (b, 0, 0)),
            pl.BlockSpec((S, D), lambda b: (0, 0)),
            pl.BlockSpec((1, 1, S), lambda b: (b, 0, 0)),
            pl.BlockSpec((1, D), lambda b: (0, 0)),
            pl.BlockSpec((1, D), lambda b: (0, 0)),
            pl.BlockSpec((D, D), lambda b: (0, 0)),
            pl.BlockSpec((1, D), lambda b: (0, 0)),
            pl.BlockSpec((D, D), lambda b: (0, 0)),
            pl.BlockSpec((1, D), lambda b: (0, 0)),
            pl.BlockSpec((D, D), lambda b: (0, 0)),
            pl.BlockSpec((1, D), lambda b: (0, 0)),
            pl.BlockSpec((D, D), lambda b: (0, 0)),
            pl.BlockSpec((1, D), lambda b: (0, 0)),
            pl.BlockSpec((1, D), lambda b: (0, 0)),
            pl.BlockSpec((1, D), lambda b: (0, 0)),
        ],
        out_specs=pl.BlockSpec((1, S, D), lambda b: (b, 0, 0)),
        out_shape=jax.ShapeDtypeStruct((B, S, D), f32),
    )(etok.reshape(B, S, D), pos_emb,
      attention_mask.astype(i32).reshape(B, 1, S),
      ln_e_g.reshape(1, D), ln_e_b.reshape(1, D),
      Wq, bq.reshape(1, D), Wk, bk.reshape(1, D), Wv, bv.reshape(1, D),
      Wo, bo.reshape(1, D), ln1_g.reshape(1, D), ln1_b.reshape(1, D))

    cls_feat = pl.pallas_call(
        _tc_ffn_body,
        grid=(B, FF // D),
        in_specs=[
            pl.BlockSpec((1, S, D), lambda b, k: (b, 0, 0)),
            pl.BlockSpec((D, D), lambda b, k: (0, k)),
            pl.BlockSpec((1, D), lambda b, k: (0, k)),
            pl.BlockSpec((D, D), lambda b, k: (k, 0)),
            pl.BlockSpec((1, D), lambda b, k: (0, 0)),
            pl.BlockSpec((1, D), lambda b, k: (0, 0)),
            pl.BlockSpec((1, D), lambda b, k: (0, 0)),
            pl.BlockSpec((D, D), lambda b, k: (0, 0)),
            pl.BlockSpec((1, D), lambda b, k: (0, 0)),
        ],
        out_specs=pl.BlockSpec((1, 1, D), lambda b, k: (b, 0, 0)),
        out_shape=jax.ShapeDtypeStruct((B, 1, D), f32),
        scratch_shapes=[pltpu.VMEM((S, D), f32)],
    )(hb1, W1, b1.reshape(1, FF), W2, b2.reshape(1, D),
      ln2_g.reshape(1, D), ln2_b.reshape(1, D),
      pool_W, pool_b.reshape(1, D)).reshape(B, D)

    w2p = jnp.zeros((128, 128), f32).at[:, :2].set(cls_W2)
    b2p = jnp.zeros((1, 128), f32).at[:, :2].set(cls_b2)
    out = pl.pallas_call(
        _tc_cls_body,
        in_specs=[
            pl.BlockSpec((NG, HID), lambda: (0, 0)),
            pl.BlockSpec((NG, 128), lambda: (0, 0)),
            pl.BlockSpec((B, D), lambda: (0, 0)),
            pl.BlockSpec((HID + D, 128), lambda: (0, 0)),
            pl.BlockSpec((1, 128), lambda: (0, 0)),
            pl.BlockSpec((128, 128), lambda: (0, 0)),
            pl.BlockSpec((1, 128), lambda: (0, 0)),
        ],
        out_specs=pl.BlockSpec((B, 128), lambda: (0, 0)),
        out_shape=jax.ShapeDtypeStruct((B, 128), f32),
    )(seg, cnt, cls_feat, cls_W1, cls_b1.reshape(1, 128), w2p, b2p)
    return out[:, :2]


# speedup vs baseline: 6.2454x; 1.3641x over previous
import functools

import jax
import jax.numpy as jnp
from jax import lax
from jax.experimental import pallas as pl
from jax.experimental.pallas import tpu as pltpu
from jax.experimental.pallas import tpu_sc as plsc

N_NODES = 10000
HID = 256
B = 32
S = 128
D = 768
H = 12
DH = 64
FF = 3072
NG = 32
N_EDGES = 160000

NC = 2
NS = 16
NW = NC * NS

RT = 640
R_ACC = RT * NS
E_TOT = N_EDGES + N_NODES
KE = 128
EB_TILE = 84
E_PAD = NS * KE * EB_TILE
EB_DEG = E_PAD // (NW * KE)
KN = 64
NODE_PB = 5
N_PAD = NW * KN * NODE_PB
KT = 64
TOK_PB = 2

_SC_MESH = plsc.VectorSubcoreMesh(core_axis_name="c", subcore_axis_name="s")


def _ln(x, g, b, eps=1e-12):
    m = x.mean(-1, keepdims=True)
    v = jnp.mean((x - m) ** 2, -1, keepdims=True)
    return (x - m) / jnp.sqrt(v + eps) * g + b


@functools.partial(
    pl.kernel,
    out_type=(
        jax.ShapeDtypeStruct((N_PAD, HID), jnp.float32),
        jax.ShapeDtypeStruct((B * S, D), jnp.float32),
    ),
    mesh=_SC_MESH,
    scratch_types=[
        pltpu.VMEM((KN,), jnp.int32),
        pltpu.VMEM((KN, HID), jnp.float32),
        pltpu.VMEM((KT,), jnp.int32),
        pltpu.VMEM((KT, D), jnp.float32),
        pltpu.SemaphoreType.DMA,
    ],
)
def _sc_gather(nidx_hbm, node_emb, tids_hbm, tok_emb, xe_out, tok_out,
               nidx_v, nrows_v, tidx_v, trows_v, sem):
    c = lax.axis_index("c")
    t = lax.axis_index("s")
    w = t * NC + c

    def node_blk(i, _):
        base = pl.multiple_of((w * NODE_PB + i) * KN, 8)
        pltpu.sync_copy(nidx_hbm.at[pl.ds(base, KN)], nidx_v)
        pltpu.async_copy(node_emb.at[nidx_v], nrows_v, sem).wait()
        pltpu.sync_copy(nrows_v, xe_out.at[pl.ds(base, KN)])
        return _
    lax.fori_loop(0, NODE_PB, node_blk, None)

    def tok_blk(i, _):
        base = pl.multiple_of((w * TOK_PB + i) * KT, 8)
        pltpu.sync_copy(tids_hbm.at[pl.ds(base, KT)], tidx_v)
        pltpu.async_copy(tok_emb.at[tidx_v], trows_v, sem).wait()
        pltpu.sync_copy(trows_v, tok_out.at[pl.ds(base, KT)])
        return _
    lax.fori_loop(0, TOK_PB, tok_blk, None)


@functools.partial(
    pl.kernel,
    out_type=jax.ShapeDtypeStruct((NC, R_ACC, 128), jnp.float32),
    mesh=_SC_MESH,
    scratch_types=[
        pltpu.VMEM((KE,), jnp.int32),
        pltpu.VMEM((KE,), jnp.int32),
        pltpu.VMEM((KE,), jnp.int32),
        pltpu.VMEM((KE,), jnp.int32),
        pltpu.VMEM((KE, 128), jnp.float32),
        pltpu.VMEM((KE, 128), jnp.float32),
        pltpu.VMEM_SHARED((R_ACC, 128), jnp.float32),
        pltpu.SemaphoreType.DMA,
        pltpu.SemaphoreType.DMA,
    ],
)
def _sc_msg_pass(hs_hbm, src2_hbm, dst_hbm, z128_hbm, scat_out,
                 s0_v, d0_v, s1_v, d1_v, rows0_v, rows1_v, acc, sem0, sem1):
    c = lax.axis_index("c")
    t = lax.axis_index("s")
    pltpu.sync_copy(z128_hbm, rows0_v)
    for j in range(RT // KE):
        base = pl.multiple_of(t * RT + j * KE, 8)
        pltpu.sync_copy(rows0_v, acc.at[pl.ds(base, KE)])
    plsc.subcore_barrier()

    def edge_pair(i, _):
        b0 = pl.multiple_of(t * (KE * EB_TILE) + (2 * i) * KE, 8)
        b1 = pl.multiple_of(t * (KE * EB_TILE) + (2 * i + 1) * KE, 8)
        pltpu.sync_copy(src2_hbm.at[c, pl.ds(b0, KE)], s0_v)
        pltpu.sync_copy(dst_hbm.at[pl.ds(b0, KE)], d0_v)
        g0 = pltpu.async_copy(hs_hbm.at[s0_v], rows0_v, sem0)
        pltpu.sync_copy(src2_hbm.at[c, pl.ds(b1, KE)], s1_v)
        pltpu.sync_copy(dst_hbm.at[pl.ds(b1, KE)], d1_v)
        g1 = pltpu.async_copy(hs_hbm.at[s1_v], rows1_v, sem1)
        g0.wait()
        pltpu.sync_copy(rows0_v, acc.at[d0_v], add=True)
        g1.wait()
        pltpu.sync_copy(rows1_v, acc.at[d1_v], add=True)
        return _
    lax.fori_loop(0, EB_TILE // 2, edge_pair, None)

    plsc.subcore_barrier()
    for j in range(RT // KE):
        base = pl.multiple_of(t * RT + j * KE, 8)
        pltpu.sync_copy(acc.at[pl.ds(base, KE)], rows0_v)
        pltpu.sync_copy(rows0_v, scat_out.at[c, pl.ds(base, KE)])


@functools.partial(
    pl.kernel,
    out_type=jax.ShapeDtypeStruct((NC, R_ACC, 128), jnp.float32),
    mesh=_SC_MESH,
    scratch_types=[
        pltpu.VMEM((KE,), jnp.int32),
        pltpu.VMEM((KE,), jnp.int32),
        pltpu.VMEM((KE, 128), jnp.float32),
        pltpu.VMEM_SHARED((R_ACC, 128), jnp.float32),
    ],
)
def _sc_deg(dst_hbm, ones_hbm, z128_hbm, deg_out, d0_v, d1_v, ones_v, acc):
    c = lax.axis_index("c")
    t = lax.axis_index("s")
    pltpu.sync_copy(z128_hbm, ones_v)
    for j in range(RT // KE):
        base = pl.multiple_of(t * RT + j * KE, 8)
        pltpu.sync_copy(ones_v, acc.at[pl.ds(base, KE)])
    pltpu.sync_copy(ones_hbm, ones_v)
    plsc.subcore_barrier()

    def edge_pair(i, _):
        b0 = pl.multiple_of(t * (KE * EB_TILE) + (2 * i) * KE, 8)
        b1 = pl.multiple_of(t * (KE * EB_TILE) + (2 * i + 1) * KE, 8)
        pltpu.sync_copy(dst_hbm.at[pl.ds(b0, KE)], d0_v)
        pltpu.sync_copy(dst_hbm.at[pl.ds(b1, KE)], d1_v)
        pltpu.sync_copy(ones_v, acc.at[d0_v], add=True)
        pltpu.sync_copy(ones_v, acc.at[d1_v], add=True)
        return _
    lax.fori_loop(0, EB_TILE // 2, edge_pair, None)

    plsc.subcore_barrier()
    for j in range(RT // KE):
        base = pl.multiple_of(t * RT + j * KE, 8)
        pltpu.sync_copy(acc.at[pl.ds(base, KE)], ones_v)
        pltpu.sync_copy(ones_v, deg_out.at[c, pl.ds(base, KE)])


BM = 1000


def _tc_hs1_body(xe_ref, w_ref, deg_ref, out_ref):
    dinv = lax.rsqrt(deg_ref[0][:, :1])
    h = jnp.dot(xe_ref[...], w_ref[...], preferred_element_type=jnp.float32)
    out_ref[...] = (h * dinv)[None]


def _tc_hs2_body(scat_ref, deg_ref, b1_ref, w_ref, out_ref):
    dinv = lax.rsqrt(deg_ref[0][:, :1])
    a = jnp.concatenate([scat_ref[0], scat_ref[1]], axis=1)
    h1 = jax.nn.relu(a * dinv + b1_ref[...])
    h = jnp.dot(h1, w_ref[...], preferred_element_type=jnp.float32)
    out_ref[...] = (h * dinv)[None]


def _tc_pool_body(scat_ref, deg_ref, b2_ref, batch_ref, seg_ref, cnt_ref):
    m = pl.program_id(0)
    dinv = lax.rsqrt(deg_ref[0][:, :1])
    a = jnp.concatenate([scat_ref[0], scat_ref[1]], axis=1)
    h2 = jax.nn.relu(a * dinv + b2_ref[...])
    bvec = batch_ref[0, 0, :]
    onehot = (bvec[:, None] == lax.broadcasted_iota(jnp.int32, (BM, NG), 1))
    onehot = onehot.astype(jnp.float32)
    contrib = lax.dot_general(onehot, h2, (((0,), (0,)), ((), ())),
                              preferred_element_type=jnp.float32)
    cntc = lax.dot_general(onehot, jnp.ones((BM, 128), jnp.float32),
                           (((0,), (0,)), ((), ())),
                           preferred_element_type=jnp.float32)

    @pl.when(m == 0)
    def _():
        seg_ref[...] = contrib
        cnt_ref[...] = cntc

    @pl.when(m > 0)
    def _():
        seg_ref[...] += contrib
        cnt_ref[...] += cntc


def _tc_attn_body(e_ref, pos_ref, mask_ref, lng_ref, lnb_ref,
                  wq_ref, bq_ref, wk_ref, bk_ref, wv_ref, bv_ref,
                  wo_ref, bo_ref, l1g_ref, l1b_ref, out_ref):
    x = e_ref[0] + pos_ref[...]
    hb0 = _ln(x, lng_ref[...], lnb_ref[...])
    q = jnp.dot(hb0, wq_ref[...], preferred_element_type=jnp.float32) + bq_ref[...]
    k = jnp.dot(hb0, wk_ref[...], preferred_element_type=jnp.float32) + bk_ref[...]
    v = jnp.dot(hb0, wv_ref[...], preferred_element_type=jnp.float32) + bv_ref[...]
    bias = jnp.where(mask_ref[0] > 0, 0.0, -1e9)
    scale = 1.0 / (DH ** 0.5)
    parts = []
    for h in range(H):
        qh = q[:, h * DH:(h + 1) * DH]
        kh = k[:, h * DH:(h + 1) * DH]
        vh = v[:, h * DH:(h + 1) * DH]
        s = lax.dot_general(qh, kh, (((1,), (1,)), ((), ())),
                            preferred_element_type=jnp.float32) * scale + bias
        att = jax.nn.softmax(s, axis=-1)
        parts.append(jnp.dot(att, vh, preferred_element_type=jnp.float32))
    ao = jnp.concatenate(parts, axis=1)
    proj = jnp.dot(ao, wo_ref[...], preferred_element_type=jnp.float32) + bo_ref[...]
    out_ref[...] = _ln(hb0 + proj, l1g_ref[...], l1b_ref[...])[None]


def _tc_ffn_body(hb_ref, w1_ref, b1_ref, w2_ref, b2_ref, l2g_ref, l2b_ref,
                 pw_ref, pb_ref, out_ref, acc_ref):
    kk = pl.program_id(1)
    x = hb_ref[0]
    tmid = jax.nn.gelu(jnp.dot(x, w1_ref[...],
                               preferred_element_type=jnp.float32) + b1_ref[...])
    c = jnp.dot(tmid, w2_ref[...], preferred_element_type=jnp.float32)

    @pl.when(kk == 0)
    def _():
        acc_ref[...] = c

    @pl.when(kk > 0)
    def _():
        acc_ref[...] += c

    @pl.when(kk == FF // D - 1)
    def _():
        hb2 = _ln(x + acc_ref[...] + b2_ref[...], l2g_ref[...], l2b_ref[...])
        cls_row = hb2[0:1, :]
        pooled = jnp.tanh(jnp.dot(cls_row, pw_ref[...],
                                  preferred_element_type=jnp.float32) + pb_ref[...])
        out_ref[...] = pooled[None]


def _tc_cls_body(seg_ref, cnt_ref, cls_ref, w1_ref, b1_ref, w2_ref, b2_ref,
                 out_ref):
    gf = seg_ref[...] / jnp.maximum(cnt_ref[:, :1], 1.0)
    fused = jnp.concatenate([gf, cls_ref[...]], axis=1)
    hid = jax.nn.relu(jnp.dot(fused, w1_ref[...],
                              preferred_element_type=jnp.float32) + b1_ref[...])
    out_ref[...] = jnp.dot(hid, w2_ref[...],
                           preferred_element_type=jnp.float32) + b2_ref[...]


def kernel(input_ids, attention_mask, x, edge_index, batch, node_emb, conv1_W,
           conv1_b, conv2_W, conv2_b, tok_emb, pos_emb, ln_e_g, ln_e_b, Wq, bq,
           Wk, bk, Wv, bv, Wo, bo, ln1_g, ln1_b, W1, b1, W2, b2, ln2_g, ln2_b,
           pool_W, pool_b, cls_W1, cls_b1, cls_W2, cls_b2):
    f32 = jnp.float32
    i32 = jnp.int32

    loop = jnp.arange(N_NODES, dtype=i32)
    src_f = jnp.concatenate([edge_index[0].astype(i32), loop])
    dst_f = jnp.concatenate([edge_index[1].astype(i32), loop])
    pad = E_PAD - E_TOT
    src_p = jnp.concatenate([src_f, jnp.zeros((pad,), i32)])
    dst_p = jnp.concatenate([dst_f, jnp.full((pad,), N_NODES, i32)])
    src2 = jnp.stack([src_p, src_p + N_NODES])
    nidx = jnp.concatenate([x[:, 0].astype(i32),
                            jnp.zeros((N_PAD - N_NODES,), i32)])
    tids = input_ids.reshape(B * S).astype(i32)
    z128 = jnp.zeros((KE, 128), f32)
    ones128 = jnp.ones((KE, 128), f32)

    xe, etok = _sc_gather(nidx, node_emb, tids, tok_emb)
    deg2 = _sc_deg(dst_p, ones128, z128)

    hs1 = pl.pallas_call(
        _tc_hs1_body,
        grid=(N_NODES // BM, 2),
        in_specs=[
            pl.BlockSpec((BM, HID), lambda m, n: (m, 0)),
            pl.BlockSpec((HID, 128), lambda m, n: (0, n)),
            pl.BlockSpec((1, BM, 128), lambda m, n: (0, m, 0)),
        ],
        out_specs=pl.BlockSpec((1, BM, 128), lambda m, n: (n, m, 0)),
        out_shape=jax.ShapeDtypeStruct((NC, N_NODES, 128), f32),
    )(xe, conv1_W, deg2)

    scat1 = _sc_msg_pass(hs1.reshape(NC * N_NODES, 128), src2, dst_p, z128)

    hs2 = pl.pallas_call(
        _tc_hs2_body,
        grid=(N_NODES // BM, 2),
        in_specs=[
            pl.BlockSpec((NC, BM, 128), lambda m, n: (0, m, 0)),
            pl.BlockSpec((1, BM, 128), lambda m, n: (0, m, 0)),
            pl.BlockSpec((1, HID), lambda m, n: (0, 0)),
            pl.BlockSpec((HID, 128), lambda m, n: (0, n)),
        ],
        out_specs=pl.BlockSpec((1, BM, 128), lambda m, n: (n, m, 0)),
        out_shape=jax.ShapeDtypeStruct((NC, N_NODES, 128), f32),
    )(scat1, deg2, conv1_b.reshape(1, HID), conv2_W)

    scat2 = _sc_msg_pass(hs2.reshape(NC * N_NODES, 128), src2, dst_p, z128)

    batch3 = batch.astype(i32).reshape(N_NODES // BM, 1, BM)
    seg, cnt = pl.pallas_call(
        _tc_pool_body,
        grid=(N_NODES // BM,),
        in_specs=[
            pl.BlockSpec((NC, BM, 128), lambda m: (0, m, 0)),
            pl.BlockSpec((1, BM, 128), lambda m: (0, m, 0)),
            pl.BlockSpec((1, HID), lambda m: (0, 0)),
            pl.BlockSpec((1, 1, BM), lambda m: (m, 0, 0)),
        ],
        out_specs=(pl.BlockSpec((NG, HID), lambda m: (0, 0)),
                   pl.BlockSpec((NG, 128), lambda m: (0, 0))),
        out_shape=(jax.ShapeDtypeStruct((NG, HID), f32),
                   jax.ShapeDtypeStruct((NG, 128), f32)),
    )(scat2, deg2, conv2_b.reshape(1, HID), batch3)

    hb1 = pl.pallas_call(
        _tc_attn_body,
        grid=(B,),
        in_specs=[
            pl.BlockSpec((1, S, D), lambda b: (b, 0, 0)),
            pl.BlockSpec((S, D), lambda b: (0, 0)),
            pl.BlockSpec((1, 1, S), lambda b: (b, 0, 0)),
            pl.BlockSpec((1, D), lambda b: (0, 0)),
            pl.BlockSpec((1, D), lambda b: (0, 0)),
            pl.BlockSpec((D, D), lambda b: (0, 0)),
            pl.BlockSpec((1, D), lambda b: (0, 0)),
            pl.BlockSpec((D, D), lambda b: (0, 0)),
            pl.BlockSpec((1, D), lambda b: (0, 0)),
            pl.BlockSpec((D, D), lambda b: (0, 0)),
            pl.BlockSpec((1, D), lambda b: (0, 0)),
            pl.BlockSpec((D, D), lambda b: (0, 0)),
            pl.BlockSpec((1, D), lambda b: (0, 0)),
            pl.BlockSpec((1, D), lambda b: (0, 0)),
            pl.BlockSpec((1, D), lambda b: (0, 0)),
        ],
        out_specs=pl.BlockSpec((1, S, D), lambda b: (b, 0, 0)),
        out_shape=jax.ShapeDtypeStruct((B, S, D), f32),
    )(etok.reshape(B, S, D), pos_emb,
      attention_mask.astype(i32).reshape(B, 1, S),
      ln_e_g.reshape(1, D), ln_e_b.reshape(1, D),
      Wq, bq.reshape(1, D), Wk, bk.reshape(1, D), Wv, bv.reshape(1, D),
      Wo, bo.reshape(1, D), ln1_g.reshape(1, D), ln1_b.reshape(1, D))

    cls_feat = pl.pallas_call(
        _tc_ffn_body,
        grid=(B, FF // D),
        in_specs=[
            pl.BlockSpec((1, S, D), lambda b, k: (b, 0, 0)),
            pl.BlockSpec((D, D), lambda b, k: (0, k)),
            pl.BlockSpec((1, D), lambda b, k: (0, k)),
            pl.BlockSpec((D, D), lambda b, k: (k, 0)),
            pl.BlockSpec((1, D), lambda b, k: (0, 0)),
            pl.BlockSpec((1, D), lambda b, k: (0, 0)),
            pl.BlockSpec((1, D), lambda b, k: (0, 0)),
            pl.BlockSpec((D, D), lambda b, k: (0, 0)),
            pl.BlockSpec((1, D), lambda b, k: (0, 0)),
        ],
        out_specs=pl.BlockSpec((1, 1, D), lambda b, k: (b, 0, 0)),
        out_shape=jax.ShapeDtypeStruct((B, 1, D), f32),
        scratch_shapes=[pltpu.VMEM((S, D), f32)],
    )(hb1, W1, b1.reshape(1, FF), W2, b2.reshape(1, D),
      ln2_g.reshape(1, D), ln2_b.reshape(1, D),
      pool_W, pool_b.reshape(1, D)).reshape(B, D)

    w2p = jnp.zeros((128, 128), f32).at[:, :2].set(cls_W2)
    b2p = jnp.zeros((1, 128), f32).at[:, :2].set(cls_b2)
    out = pl.pallas_call(
        _tc_cls_body,
        in_specs=[
            pl.BlockSpec((NG, HID), lambda: (0, 0)),
            pl.BlockSpec((NG, 128), lambda: (0, 0)),
            pl.BlockSpec((B, D), lambda: (0, 0)),
            pl.BlockSpec((HID + D, 128), lambda: (0, 0)),
            pl.BlockSpec((1, 128), lambda: (0, 0)),
            pl.BlockSpec((128, 128), lambda: (0, 0)),
            pl.BlockSpec((1, 128), lambda: (0, 0)),
        ],
        out_specs=pl.BlockSpec((B, 128), lambda: (0, 0)),
        out_shape=jax.ShapeDtypeStruct((B, 128), f32),
    )(seg, cnt, cls_feat, cls_W1, cls_b1.reshape(1, 128), w2p, b2p)
    return out[:, :2]


# speedup vs baseline: 6.7465x; 1.0802x over previous
import functools

import jax
import jax.numpy as jnp
from jax import lax
from jax.experimental import pallas as pl
from jax.experimental.pallas import tpu as pltpu
from jax.experimental.pallas import tpu_sc as plsc

N_NODES = 10000
HID = 256
B = 32
S = 128
D = 768
H = 12
DH = 64
FF = 3072
NG = 32
N_EDGES = 160000

NC = 2
NS = 16
NW = NC * NS

RT = 640
R_ACC = RT * NS
E_TOT = N_EDGES + N_NODES
KE = 128
EB_TILE = 84
E_PAD = NS * KE * EB_TILE
EB_DEG = E_PAD // (NW * KE)
KN = 64
NODE_PB = 5
N_PAD = NW * KN * NODE_PB
KT = 64
TOK_PB = 2

_SC_MESH = plsc.VectorSubcoreMesh(core_axis_name="c", subcore_axis_name="s")


def _ln(x, g, b, eps=1e-12):
    m = x.mean(-1, keepdims=True)
    v = jnp.mean((x - m) ** 2, -1, keepdims=True)
    return (x - m) / jnp.sqrt(v + eps) * g + b


@functools.partial(
    pl.kernel,
    out_type=(
        jax.ShapeDtypeStruct((N_PAD, HID), jnp.float32),
        jax.ShapeDtypeStruct((B * S, D), jnp.float32),
    ),
    mesh=_SC_MESH,
    scratch_types=[
        pltpu.VMEM((KN,), jnp.int32),
        pltpu.VMEM((KN, HID), jnp.float32),
        pltpu.VMEM((KT,), jnp.int32),
        pltpu.VMEM((KT, D), jnp.float32),
        pltpu.SemaphoreType.DMA,
    ],
)
def _sc_gather(nidx_hbm, node_emb, tids_hbm, tok_emb, xe_out, tok_out,
               nidx_v, nrows_v, tidx_v, trows_v, sem):
    c = lax.axis_index("c")
    t = lax.axis_index("s")
    w = t * NC + c

    def node_blk(i, _):
        base = pl.multiple_of((w * NODE_PB + i) * KN, 8)
        pltpu.sync_copy(nidx_hbm.at[pl.ds(base, KN)], nidx_v)
        pltpu.async_copy(node_emb.at[nidx_v], nrows_v, sem).wait()
        pltpu.sync_copy(nrows_v, xe_out.at[pl.ds(base, KN)])
        return _
    lax.fori_loop(0, NODE_PB, node_blk, None)

    def tok_blk(i, _):
        base = pl.multiple_of((w * TOK_PB + i) * KT, 8)
        pltpu.sync_copy(tids_hbm.at[pl.ds(base, KT)], tidx_v)
        pltpu.async_copy(tok_emb.at[tidx_v], trows_v, sem).wait()
        pltpu.sync_copy(trows_v, tok_out.at[pl.ds(base, KT)])
        return _
    lax.fori_loop(0, TOK_PB, tok_blk, None)


@functools.partial(
    pl.kernel,
    out_type=jax.ShapeDtypeStruct((NC, R_ACC, 128), jnp.float32),
    mesh=_SC_MESH,
    scratch_types=[
        pltpu.VMEM((KE,), jnp.int32),
        pltpu.VMEM((KE,), jnp.int32),
        pltpu.VMEM((KE,), jnp.int32),
        pltpu.VMEM((KE,), jnp.int32),
        pltpu.VMEM((KE, 128), jnp.float32),
        pltpu.VMEM((KE, 128), jnp.float32),
        pltpu.VMEM_SHARED((R_ACC, 128), jnp.float32),
        pltpu.SemaphoreType.DMA,
        pltpu.SemaphoreType.DMA,
        pltpu.SemaphoreType.DMA,
        pltpu.SemaphoreType.DMA,
    ],
)
def _sc_msg_pass(hs_hbm, src2_hbm, dst_hbm, z128_hbm, scat_out,
                 s0_v, d0_v, s1_v, d1_v, rows0_v, rows1_v, acc,
                 sem0, sem1, ssem0, ssem1):
    c = lax.axis_index("c")
    t = lax.axis_index("s")
    pltpu.sync_copy(z128_hbm, rows0_v)
    for j in range(RT // KE):
        base = pl.multiple_of(t * RT + j * KE, 8)
        pltpu.sync_copy(rows0_v, acc.at[pl.ds(base, KE)])
    plsc.subcore_barrier()

    def edge_pair(i, _):
        b0 = pl.multiple_of(t * (KE * EB_TILE) + (2 * i) * KE, 8)
        b1 = pl.multiple_of(t * (KE * EB_TILE) + (2 * i + 1) * KE, 8)

        @pl.when(i > 0)
        def _():
            pltpu.make_async_copy(rows0_v, acc.at[d0_v], ssem0).wait()
        pltpu.sync_copy(src2_hbm.at[c, pl.ds(b0, KE)], s0_v)
        pltpu.sync_copy(dst_hbm.at[pl.ds(b0, KE)], d0_v)
        g0 = pltpu.async_copy(hs_hbm.at[s0_v], rows0_v, sem0)

        @pl.when(i > 0)
        def _():
            pltpu.make_async_copy(rows1_v, acc.at[d1_v], ssem1).wait()
        pltpu.sync_copy(src2_hbm.at[c, pl.ds(b1, KE)], s1_v)
        pltpu.sync_copy(dst_hbm.at[pl.ds(b1, KE)], d1_v)
        g1 = pltpu.async_copy(hs_hbm.at[s1_v], rows1_v, sem1)

        g0.wait()
        pltpu.async_copy(rows0_v, acc.at[d0_v], ssem0, add=True)
        g1.wait()
        pltpu.async_copy(rows1_v, acc.at[d1_v], ssem1, add=True)
        return _
    lax.fori_loop(0, EB_TILE // 2, edge_pair, None)

    pltpu.make_async_copy(rows0_v, acc.at[d0_v], ssem0).wait()
    pltpu.make_async_copy(rows1_v, acc.at[d1_v], ssem1).wait()
    plsc.subcore_barrier()
    for j in range(RT // KE):
        base = pl.multiple_of(t * RT + j * KE, 8)
        pltpu.sync_copy(acc.at[pl.ds(base, KE)], rows0_v)
        pltpu.sync_copy(rows0_v, scat_out.at[c, pl.ds(base, KE)])


@functools.partial(
    pl.kernel,
    out_type=jax.ShapeDtypeStruct((NC, R_ACC, 128), jnp.float32),
    mesh=_SC_MESH,
    scratch_types=[
        pltpu.VMEM((KE,), jnp.int32),
        pltpu.VMEM((KE,), jnp.int32),
        pltpu.VMEM((KE, 128), jnp.float32),
        pltpu.VMEM_SHARED((R_ACC, 128), jnp.float32),
        pltpu.SemaphoreType.DMA,
        pltpu.SemaphoreType.DMA,
    ],
)
def _sc_deg(dst_hbm, ones_hbm, z128_hbm, deg_out, d0_v, d1_v, ones_v, acc,
            ssem0, ssem1):
    c = lax.axis_index("c")
    t = lax.axis_index("s")
    pltpu.sync_copy(z128_hbm, ones_v)
    for j in range(RT // KE):
        base = pl.multiple_of(t * RT + j * KE, 8)
        pltpu.sync_copy(ones_v, acc.at[pl.ds(base, KE)])
    pltpu.sync_copy(ones_hbm, ones_v)
    plsc.subcore_barrier()

    def edge_pair(i, _):
        b0 = pl.multiple_of(t * (KE * EB_TILE) + (2 * i) * KE, 8)
        b1 = pl.multiple_of(t * (KE * EB_TILE) + (2 * i + 1) * KE, 8)

        @pl.when(i > 0)
        def _():
            pltpu.make_async_copy(ones_v, acc.at[d0_v], ssem0).wait()
        pltpu.sync_copy(dst_hbm.at[pl.ds(b0, KE)], d0_v)
        pltpu.async_copy(ones_v, acc.at[d0_v], ssem0, add=True)

        @pl.when(i > 0)
        def _():
            pltpu.make_async_copy(ones_v, acc.at[d1_v], ssem1).wait()
        pltpu.sync_copy(dst_hbm.at[pl.ds(b1, KE)], d1_v)
        pltpu.async_copy(ones_v, acc.at[d1_v], ssem1, add=True)
        return _
    lax.fori_loop(0, EB_TILE // 2, edge_pair, None)

    pltpu.make_async_copy(ones_v, acc.at[d0_v], ssem0).wait()
    pltpu.make_async_copy(ones_v, acc.at[d1_v], ssem1).wait()
    plsc.subcore_barrier()
    for j in range(RT // KE):
        base = pl.multiple_of(t * RT + j * KE, 8)
        pltpu.sync_copy(acc.at[pl.ds(base, KE)], ones_v)
        pltpu.sync_copy(ones_v, deg_out.at[c, pl.ds(base, KE)])


BM = 1000


def _tc_hs1_body(xe_ref, w_ref, deg_ref, out_ref):
    dinv = lax.rsqrt(deg_ref[0][:, :1])
    h = jnp.dot(xe_ref[...], w_ref[...], preferred_element_type=jnp.float32)
    out_ref[...] = (h * dinv)[None]


def _tc_hs2_body(scat_ref, deg_ref, b1_ref, w_ref, out_ref):
    dinv = lax.rsqrt(deg_ref[0][:, :1])
    a = jnp.concatenate([scat_ref[0], scat_ref[1]], axis=1)
    h1 = jax.nn.relu(a * dinv + b1_ref[...])
    h = jnp.dot(h1, w_ref[...], preferred_element_type=jnp.float32)
    out_ref[...] = (h * dinv)[None]


def _tc_pool_body(scat_ref, deg_ref, b2_ref, batch_ref, seg_ref, cnt_ref):
    m = pl.program_id(0)
    dinv = lax.rsqrt(deg_ref[0][:, :1])
    a = jnp.concatenate([scat_ref[0], scat_ref[1]], axis=1)
    h2 = jax.nn.relu(a * dinv + b2_ref[...])
    bvec = batch_ref[0, 0, :]
    onehot = (bvec[:, None] == lax.broadcasted_iota(jnp.int32, (BM, NG), 1))
    onehot = onehot.astype(jnp.float32)
    contrib = lax.dot_general(onehot, h2, (((0,), (0,)), ((), ())),
                              preferred_element_type=jnp.float32)
    cntc = lax.dot_general(onehot, jnp.ones((BM, 128), jnp.float32),
                           (((0,), (0,)), ((), ())),
                           preferred_element_type=jnp.float32)

    @pl.when(m == 0)
    def _():
        seg_ref[...] = contrib
        cnt_ref[...] = cntc

    @pl.when(m > 0)
    def _():
        seg_ref[...] += contrib
        cnt_ref[...] += cntc


def _tc_attn_body(e_ref, pos_ref, mask_ref, lng_ref, lnb_ref,
                  wq_ref, bq_ref, wk_ref, bk_ref, wv_ref, bv_ref,
                  wo_ref, bo_ref, l1g_ref, l1b_ref, out_ref):
    x = e_ref[0] + pos_ref[...]
    hb0 = _ln(x, lng_ref[...], lnb_ref[...])
    q = jnp.dot(hb0, wq_ref[...], preferred_element_type=jnp.float32) + bq_ref[...]
    k = jnp.dot(hb0, wk_ref[...], preferred_element_type=jnp.float32) + bk_ref[...]
    v = jnp.dot(hb0, wv_ref[...], preferred_element_type=jnp.float32) + bv_ref[...]
    bias = jnp.where(mask_ref[0] > 0, 0.0, -1e9)
    scale = 1.0 / (DH ** 0.5)
    parts = []
    for h in range(H):
        qh = q[:, h * DH:(h + 1) * DH]
        kh = k[:, h * DH:(h + 1) * DH]
        vh = v[:, h * DH:(h + 1) * DH]
        s = lax.dot_general(qh, kh, (((1,), (1,)), ((), ())),
                            preferred_element_type=jnp.float32) * scale + bias
        att = jax.nn.softmax(s, axis=-1)
        parts.append(jnp.dot(att, vh, preferred_element_type=jnp.float32))
    ao = jnp.concatenate(parts, axis=1)
    proj = jnp.dot(ao, wo_ref[...], preferred_element_type=jnp.float32) + bo_ref[...]
    out_ref[...] = _ln(hb0 + proj, l1g_ref[...], l1b_ref[...])[None]


def _tc_ffn_body(hb_ref, w1_ref, b1_ref, w2_ref, b2_ref, l2g_ref, l2b_ref,
                 pw_ref, pb_ref, out_ref, acc_ref):
    kk = pl.program_id(1)
    x = hb_ref[0]
    tmid = jax.nn.gelu(jnp.dot(x, w1_ref[...],
                               preferred_element_type=jnp.float32) + b1_ref[...])
    c = jnp.dot(tmid, w2_ref[...], preferred_element_type=jnp.float32)

    @pl.when(kk == 0)
    def _():
        acc_ref[...] = c

    @pl.when(kk > 0)
    def _():
        acc_ref[...] += c

    @pl.when(kk == FF // D - 1)
    def _():
        hb2 = _ln(x + acc_ref[...] + b2_ref[...], l2g_ref[...], l2b_ref[...])
        cls_row = hb2[0:1, :]
        pooled = jnp.tanh(jnp.dot(cls_row, pw_ref[...],
                                  preferred_element_type=jnp.float32) + pb_ref[...])
        out_ref[...] = pooled[None]


def _tc_cls_body(seg_ref, cnt_ref, cls_ref, w1_ref, b1_ref, w2_ref, b2_ref,
                 out_ref):
    gf = seg_ref[...] / jnp.maximum(cnt_ref[:, :1], 1.0)
    fused = jnp.concatenate([gf, cls_ref[...]], axis=1)
    hid = jax.nn.relu(jnp.dot(fused, w1_ref[...],
                              preferred_element_type=jnp.float32) + b1_ref[...])
    out_ref[...] = jnp.dot(hid, w2_ref[...],
                           preferred_element_type=jnp.float32) + b2_ref[...]


def kernel(input_ids, attention_mask, x, edge_index, batch, node_emb, conv1_W,
           conv1_b, conv2_W, conv2_b, tok_emb, pos_emb, ln_e_g, ln_e_b, Wq, bq,
           Wk, bk, Wv, bv, Wo, bo, ln1_g, ln1_b, W1, b1, W2, b2, ln2_g, ln2_b,
           pool_W, pool_b, cls_W1, cls_b1, cls_W2, cls_b2):
    f32 = jnp.float32
    i32 = jnp.int32

    loop = jnp.arange(N_NODES, dtype=i32)
    src_f = jnp.concatenate([edge_index[0].astype(i32), loop])
    dst_f = jnp.concatenate([edge_index[1].astype(i32), loop])
    pad = E_PAD - E_TOT
    src_p = jnp.concatenate([src_f, jnp.zeros((pad,), i32)])
    dst_p = jnp.concatenate([dst_f, jnp.full((pad,), N_NODES, i32)])
    src2 = jnp.stack([src_p, src_p + N_NODES])
    nidx = jnp.concatenate([x[:, 0].astype(i32),
                            jnp.zeros((N_PAD - N_NODES,), i32)])
    tids = input_ids.reshape(B * S).astype(i32)
    z128 = jnp.zeros((KE, 128), f32)
    ones128 = jnp.ones((KE, 128), f32)

    xe, etok = _sc_gather(nidx, node_emb, tids, tok_emb)
    deg2 = _sc_deg(dst_p, ones128, z128)

    hs1 = pl.pallas_call(
        _tc_hs1_body,
        grid=(N_NODES // BM, 2),
        in_specs=[
            pl.BlockSpec((BM, HID), lambda m, n: (m, 0)),
            pl.BlockSpec((HID, 128), lambda m, n: (0, n)),
            pl.BlockSpec((1, BM, 128), lambda m, n: (0, m, 0)),
        ],
        out_specs=pl.BlockSpec((1, BM, 128), lambda m, n: (n, m, 0)),
        out_shape=jax.ShapeDtypeStruct((NC, N_NODES, 128), f32),
    )(xe, conv1_W, deg2)

    scat1 = _sc_msg_pass(hs1.reshape(NC * N_NODES, 128), src2, dst_p, z128)

    hs2 = pl.pallas_call(
        _tc_hs2_body,
        grid=(N_NODES // BM, 2),
        in_specs=[
            pl.BlockSpec((NC, BM, 128), lambda m, n: (0, m, 0)),
            pl.BlockSpec((1, BM, 128), lambda m, n: (0, m, 0)),
            pl.BlockSpec((1, HID), lambda m, n: (0, 0)),
            pl.BlockSpec((HID, 128), lambda m, n: (0, n)),
        ],
        out_specs=pl.BlockSpec((1, BM, 128), lambda m, n: (n, m, 0)),
        out_shape=jax.ShapeDtypeStruct((NC, N_NODES, 128), f32),
    )(scat1, deg2, conv1_b.reshape(1, HID), conv2_W)

    scat2 = _sc_msg_pass(hs2.reshape(NC * N_NODES, 128), src2, dst_p, z128)

    batch3 = batch.astype(i32).reshape(N_NODES // BM, 1, BM)
    seg, cnt = pl.pallas_call(
        _tc_pool_body,
        grid=(N_NODES // BM,),
        in_specs=[
            pl.BlockSpec((NC, BM, 128), lambda m: (0, m, 0)),
            pl.BlockSpec((1, BM, 128), lambda m: (0, m, 0)),
            pl.BlockSpec((1, HID), lambda m: (0, 0)),
            pl.BlockSpec((1, 1, BM), lambda m: (m, 0, 0)),
        ],
        out_specs=(pl.BlockSpec((NG, HID), lambda m: (0, 0)),
                   pl.BlockSpec((NG, 128), lambda m: (0, 0))),
        out_shape=(jax.ShapeDtypeStruct((NG, HID), f32),
                   jax.ShapeDtypeStruct((NG, 128), f32)),
    )(scat2, deg2, conv2_b.reshape(1, HID), batch3)

    hb1 = pl.pallas_call(
        _tc_attn_body,
        grid=(B,),
        in_specs=[
            pl.BlockSpec((1, S, D), lambda b: (b, 0, 0)),
            pl.BlockSpec((S, D), lambda b: (0, 0)),
            pl.BlockSpec((1, 1, S), lambda b: (b, 0, 0)),
            pl.BlockSpec((1, D), lambda b: (0, 0)),
            pl.BlockSpec((1, D), lambda b: (0, 0)),
            pl.BlockSpec((D, D), lambda b: (0, 0)),
            pl.BlockSpec((1, D), lambda b: (0, 0)),
            pl.BlockSpec((D, D), lambda b: (0, 0)),
            pl.BlockSpec((1, D), lambda b: (0, 0)),
            pl.BlockSpec((D, D), lambda b: (0, 0)),
            pl.BlockSpec((1, D), lambda b: (0, 0)),
            pl.BlockSpec((D, D), lambda b: (0, 0)),
            pl.BlockSpec((1, D), lambda b: (0, 0)),
            pl.BlockSpec((1, D), lambda b: (0, 0)),
            pl.BlockSpec((1, D), lambda b: (0, 0)),
        ],
        out_specs=pl.BlockSpec((1, S, D), lambda b: (b, 0, 0)),
        out_shape=jax.ShapeDtypeStruct((B, S, D), f32),
    )(etok.reshape(B, S, D), pos_emb,
      attention_mask.astype(i32).reshape(B, 1, S),
      ln_e_g.reshape(1, D), ln_e_b.reshape(1, D),
      Wq, bq.reshape(1, D), Wk, bk.reshape(1, D), Wv, bv.reshape(1, D),
      Wo, bo.reshape(1, D), ln1_g.reshape(1, D), ln1_b.reshape(1, D))

    cls_feat = pl.pallas_call(
        _tc_ffn_body,
        grid=(B, FF // D),
        in_specs=[
            pl.BlockSpec((1, S, D), lambda b, k: (b, 0, 0)),
            pl.BlockSpec((D, D), lambda b, k: (0, k)),
            pl.BlockSpec((1, D), lambda b, k: (0, k)),
            pl.BlockSpec((D, D), lambda b, k: (k, 0)),
            pl.BlockSpec((1, D), lambda b, k: (0, 0)),
            pl.BlockSpec((1, D), lambda b, k: (0, 0)),
            pl.BlockSpec((1, D), lambda b, k: (0, 0)),
            pl.BlockSpec((D, D), lambda b, k: (0, 0)),
            pl.BlockSpec((1, D), lambda b, k: (0, 0)),
        ],
        out_specs=pl.BlockSpec((1, 1, D), lambda b, k: (b, 0, 0)),
        out_shape=jax.ShapeDtypeStruct((B, 1, D), f32),
        scratch_shapes=[pltpu.VMEM((S, D), f32)],
    )(hb1, W1, b1.reshape(1, FF), W2, b2.reshape(1, D),
      ln2_g.reshape(1, D), ln2_b.reshape(1, D),
      pool_W, pool_b.reshape(1, D)).reshape(B, D)

    w2p = jnp.zeros((128, 128), f32).at[:, :2].set(cls_W2)
    b2p = jnp.zeros((1, 128), f32).at[:, :2].set(cls_b2)
    out = pl.pallas_call(
        _tc_cls_body,
        in_specs=[
            pl.BlockSpec((NG, HID), lambda: (0, 0)),
            pl.BlockSpec((NG, 128), lambda: (0, 0)),
            pl.BlockSpec((B, D), lambda: (0, 0)),
            pl.BlockSpec((HID + D, 128), lambda: (0, 0)),
            pl.BlockSpec((1, 128), lambda: (0, 0)),
            pl.BlockSpec((128, 128), lambda: (0, 0)),
            pl.BlockSpec((1, 128), lambda: (0, 0)),
        ],
        out_specs=pl.BlockSpec((B, 128), lambda: (0, 0)),
        out_shape=jax.ShapeDtypeStruct((B, 128), f32),
    )(seg, cnt, cls_feat, cls_W1, cls_b1.reshape(1, 128), w2p, b2p)
    return out[:, :2]
